# Initial kernel scaffold; baseline (speedup 1.0000x reference)
import jax
import jax.numpy as jnp
from jax.experimental import pallas as pl


def kernel(x, edge_index, edge_attr, W_fc1, b_fc1, W_g0, b_g0, W_g1, b_g1, W_fc2, b_fc2):
    raise NotImplementedError("write your pallas kernel here")


# speedup vs baseline: 6.2967x; 6.2967x over previous
import functools

import jax
import jax.numpy as jnp
from jax import lax
from jax.experimental import pallas as pl
from jax.experimental.pallas import tpu as pltpu
from jax.experimental.pallas import tpu_sc as plsc

N = 10000
E = 320000
D = 128

NC = 2
NS = 16
NW = NC * NS
K = 80
EPW = E // NW
NCHUNK = EPW // K
NPAD = 10240
NP = 3
PR = 3456
OPAD = NP * PR
NDUMP = 128
AR = PR + NDUMP
ZSTRIPE = AR // NS
OSTRIPE = PR // NS

_sc_mesh = plsc.VectorSubcoreMesh(core_axis_name="c", subcore_axis_name="s")


@functools.partial(
    pl.kernel,
    out_type=jax.ShapeDtypeStruct((NC, NPAD), jnp.float32),
    mesh=_sc_mesh,
    scratch_types=[
        pltpu.VMEM((NCHUNK, K), jnp.int32),
        pltpu.VMEM((NCHUNK, K), jnp.float32),
        pltpu.VMEM((NPAD // NS,), jnp.float32),
        pltpu.VMEM_SHARED((NPAD,), jnp.float32),
    ],
)
def _sc_degree(col3, ew3, out, colv, ewv, zb, acc):
    c = lax.axis_index("c")
    s = lax.axis_index("s")
    w = c * NS + s
    stripe = NPAD // NS

    def zb_body(i, _):
        zb[pl.ds(i * 16, 16)] = jnp.zeros((16,), jnp.float32)
        return 0

    lax.fori_loop(0, stripe // 16, zb_body, 0)
    pltpu.sync_copy(zb, acc.at[pl.ds(s * stripe, stripe)])
    pltpu.sync_copy(col3.at[w], colv)
    pltpu.sync_copy(ew3.at[w], ewv)
    plsc.subcore_barrier()

    def chunk_body(j, _):
        pltpu.sync_copy(ewv.at[j], acc.at[colv.at[j]], add=True)
        return 0

    lax.fori_loop(0, NCHUNK, chunk_body, 0)
    plsc.subcore_barrier()
    pltpu.sync_copy(acc.at[pl.ds(s * stripe, stripe)],
                    out.at[c, pl.ds(s * stripe, stripe)])


@functools.partial(
    pl.kernel,
    out_type=jax.ShapeDtypeStruct((NC, OPAD, D), jnp.float32),
    mesh=_sc_mesh,
    scratch_types=[
        pltpu.VMEM((NCHUNK, K), jnp.int32),
        pltpu.VMEM((NCHUNK, K), jnp.int32),
        pltpu.VMEM((NCHUNK, K), jnp.float32),
        pltpu.VMEM((1, K), jnp.int32),
        pltpu.VMEM((K, D), jnp.float32),
        pltpu.VMEM((ZSTRIPE, D), jnp.float32),
        pltpu.VMEM_SHARED((AR, D), jnp.float32),
        pltpu.SemaphoreType.DMA,
    ],
)
def _sc_edge_agg(g, row3, col3, ew3, out, rowv, colv, ewv, colw, gbuf, zb,
                 acc, sem):
    c = lax.axis_index("c")
    s = lax.axis_index("s")
    w = c * NS + s

    def zb_body(i, _):
        r = i // (D // 16)
        v = i % (D // 16)
        zb[r, pl.ds(v * 16, 16)] = jnp.zeros((16,), jnp.float32)
        return 0

    lax.fori_loop(0, ZSTRIPE * (D // 16), zb_body, 0)
    pltpu.sync_copy(row3.at[w], rowv)
    pltpu.sync_copy(col3.at[w], colv)
    pltpu.sync_copy(ew3.at[w], ewv)

    for p in range(NP):
        pltpu.sync_copy(zb, acc.at[pl.ds(s * ZSTRIPE, ZSTRIPE)])
        plsc.subcore_barrier()

        def chunk_body(j, _):
            pltpu.async_copy(g.at[rowv.at[j]], gbuf, sem).wait()

            for eb in range(K // 16):
                sl16 = pl.ds(eb * 16, 16)
                col16 = colv[j, sl16]
                rel = col16 - p * PR
                oob = (rel < 0) | (rel >= PR)
                dump = PR + (col16 & (NDUMP - 1))
                colw[0, sl16] = jnp.where(oob, dump, rel)

            def scale_body(eb, _):
                ew16 = ewv[j, pl.ds(eb * 16, 16)]
                for l in range(16):
                    sc = ew16[l]
                    e = eb * 16 + l
                    for v in range(D // 16):
                        sl = pl.ds(v * 16, 16)
                        gbuf[e, sl] = gbuf[e, sl] * sc
                return 0

            lax.fori_loop(0, K // 16, scale_body, 0)
            pltpu.sync_copy(gbuf, acc.at[colw.at[0]], add=True)
            return 0

        lax.fori_loop(0, NCHUNK, chunk_body, 0)
        plsc.subcore_barrier()
        pltpu.sync_copy(
            acc.at[pl.ds(s * OSTRIPE, OSTRIPE)],
            out.at[c, pl.ds(p * PR + s * OSTRIPE, OSTRIPE), :])
        plsc.subcore_barrier()


_NB = 1024
_GRID = (NPAD // _NB,)

_rows = pl.BlockSpec((_NB, D), lambda i: (i, 0))
_spart = pl.BlockSpec((NC, _NB, D), lambda i: (0, i, 0))
_full = pl.BlockSpec((D, D), lambda i: (0, 0))
_bias = pl.BlockSpec((1, D), lambda i: (0, 0))
_degs = pl.BlockSpec((NC, _NB), lambda i: (0, i))


def _dinv_of(degp):
    return lax.rsqrt(degp[0, :] + degp[1, :] + 1.0)[:, None]


def _mm_t(a, w):
    return lax.dot_general(a, w, (((1,), (1,)), ((), ())),
                           preferred_element_type=jnp.float32)


def _tc_pre(x_ref, w1_ref, b1_ref, wg0_ref, degp_ref, g0_ref):
    y1 = _mm_t(x_ref[...], w1_ref[...]) + b1_ref[...]
    g0_ref[...] = _dinv_of(degp_ref[...]) * _mm_t(y1, wg0_ref[...])


def _tc_mid(s3_ref, g0_ref, degp_ref, bg0_ref, wg1_ref, g1_ref):
    dinv = _dinv_of(degp_ref[...])
    y2 = dinv * (s3_ref[0] + s3_ref[1] + g0_ref[...]) + bg0_ref[...]
    g1_ref[...] = dinv * _mm_t(y2, wg1_ref[...])


def _tc_post(s3_ref, g1_ref, degp_ref, bg1_ref, w2_ref, b2_ref, o_ref):
    dinv = _dinv_of(degp_ref[...])
    y3 = dinv * (s3_ref[0] + s3_ref[1] + g1_ref[...]) + bg1_ref[...]
    o_ref[...] = jax.nn.sigmoid(_mm_t(y3, w2_ref[...]) + b2_ref[...])


_gshape = jax.ShapeDtypeStruct((NPAD, D), jnp.float32)

_tc_pre_call = pl.pallas_call(
    _tc_pre,
    grid=_GRID,
    in_specs=[_rows, _full, _bias, _full, _degs],
    out_specs=_rows,
    out_shape=_gshape,
)

_tc_mid_call = pl.pallas_call(
    _tc_mid,
    grid=_GRID,
    in_specs=[_spart, _rows, _degs, _bias, _full],
    out_specs=_rows,
    out_shape=_gshape,
)

_tc_post_call = pl.pallas_call(
    _tc_post,
    grid=_GRID,
    in_specs=[_spart, _rows, _degs, _bias, _full, _bias],
    out_specs=_rows,
    out_shape=jax.ShapeDtypeStruct((N, D), jnp.float32),
)


def kernel(x, edge_index, edge_attr, W_fc1, b_fc1, W_g0, b_g0, W_g1, b_g1,
           W_fc2, b_fc2):
    row3 = edge_index[0].astype(jnp.int32).reshape(NW, NCHUNK, K)
    col3 = edge_index[1].astype(jnp.int32).reshape(NW, NCHUNK, K)
    ew3 = edge_attr.astype(jnp.float32).reshape(NW, NCHUNK, K)

    degp = _sc_degree(col3, ew3)
    g0 = _tc_pre_call(x, W_fc1, b_fc1.reshape(1, D), W_g0, degp)
    s0 = _sc_edge_agg(g0, row3, col3, ew3)
    g1 = _tc_mid_call(s0, g0, degp, b_g0.reshape(1, D), W_g1)
    s1 = _sc_edge_agg(g1, row3, col3, ew3)
    return _tc_post_call(s1, g1, degp, b_g1.reshape(1, D), W_fc2,
                         b_fc2.reshape(1, D))


# speedup vs baseline: 10.0819x; 1.6012x over previous
import functools

import jax
import jax.numpy as jnp
from jax import lax
from jax.experimental import pallas as pl
from jax.experimental.pallas import tpu as pltpu
from jax.experimental.pallas import tpu_sc as plsc

N = 10000
E = 320000
D = 128

NC = 2
NS = 16
NW = NC * NS
K = 80
EPW = E // NW
NCHUNK = EPW // K
NPAD = 10240
NP = 3
PR = 3456
OPAD = NP * PR
NDUMP = 128
AR = PR + NDUMP
ZSTRIPE = AR // NS
OSTRIPE = PR // NS

_sc_mesh = plsc.VectorSubcoreMesh(core_axis_name="c", subcore_axis_name="s")


@functools.partial(
    pl.kernel,
    out_type=jax.ShapeDtypeStruct((NC, NPAD), jnp.float32),
    mesh=_sc_mesh,
    scratch_types=[
        pltpu.VMEM((NCHUNK, K), jnp.int32),
        pltpu.VMEM((NCHUNK, K), jnp.float32),
        pltpu.VMEM((NPAD // NS,), jnp.float32),
        pltpu.VMEM_SHARED((NPAD,), jnp.float32),
    ],
)
def _sc_degree(col3, ew3, out, colv, ewv, zb, acc):
    c = lax.axis_index("c")
    s = lax.axis_index("s")
    w = c * NS + s
    stripe = NPAD // NS

    def zb_body(i, _):
        zb[pl.ds(i * 16, 16)] = jnp.zeros((16,), jnp.float32)
        return 0

    lax.fori_loop(0, stripe // 16, zb_body, 0)
    pltpu.sync_copy(zb, acc.at[pl.ds(s * stripe, stripe)])
    pltpu.sync_copy(col3.at[w], colv)
    pltpu.sync_copy(ew3.at[w], ewv)
    plsc.subcore_barrier()

    def chunk_body(j, _):
        pltpu.sync_copy(ewv.at[j], acc.at[colv.at[j]], add=True)
        return 0

    lax.fori_loop(0, NCHUNK, chunk_body, 0)
    plsc.subcore_barrier()
    pltpu.sync_copy(acc.at[pl.ds(s * stripe, stripe)],
                    out.at[c, pl.ds(s * stripe, stripe)])


@functools.partial(
    pl.kernel,
    out_type=jax.ShapeDtypeStruct((NC, OPAD, D), jnp.float32),
    mesh=_sc_mesh,
    scratch_types=[
        pltpu.VMEM((NCHUNK, K), jnp.int32),
        pltpu.VMEM((NCHUNK, K), jnp.int32),
        pltpu.VMEM((NCHUNK, K), jnp.float32),
        pltpu.VMEM((1, K), jnp.int32),
        pltpu.VMEM((1, K), jnp.int32),
        pltpu.VMEM((K, D), jnp.float32),
        pltpu.VMEM((K, D), jnp.float32),
        pltpu.VMEM((ZSTRIPE, D), jnp.float32),
        pltpu.VMEM_SHARED((AR, D), jnp.float32),
        pltpu.SemaphoreType.DMA,
        pltpu.SemaphoreType.DMA,
        pltpu.SemaphoreType.DMA,
        pltpu.SemaphoreType.DMA,
    ],
)
def _sc_edge_agg(g, row3, col3, ew3, out, rowv, colv, ewv, colwa, colwb,
                 gbufa, gbufb, zb, acc, gsa, gsb, ssa, ssb):
    c = lax.axis_index("c")
    s = lax.axis_index("s")
    w = c * NS + s

    def zb_body(i, _):
        r = i // (D // 16)
        v = i % (D // 16)
        zb[r, pl.ds(v * 16, 16)] = jnp.zeros((16,), jnp.float32)
        return 0

    lax.fori_loop(0, ZSTRIPE * (D // 16), zb_body, 0)
    pltpu.sync_copy(row3.at[w], rowv)
    pltpu.sync_copy(col3.at[w], colv)
    pltpu.sync_copy(ew3.at[w], ewv)

    def _gather_start(j, buf, sem):
        pltpu.async_copy(g.at[rowv.at[j]], buf, sem)

    def _gather_wait(j, buf, sem):
        pltpu.make_async_copy(g.at[rowv.at[j]], buf, sem).wait()

    def _scatter_start(buf, colw, sem):
        pltpu.async_copy(buf, acc.at[colw.at[0]], sem, add=True)

    def _scatter_wait(buf, colw, sem):
        pltpu.make_async_copy(buf, acc.at[colw.at[0]], sem).wait()

    for p in range(NP):

        def _process(j, buf, colw):
            for eb in range(K // 16):
                sl16 = pl.ds(eb * 16, 16)
                col16 = colv[j, sl16]
                rel = col16 - p * PR
                oob = (rel < 0) | (rel >= PR)
                dump = PR + (col16 & (NDUMP - 1))
                colw[0, sl16] = jnp.where(oob, dump, rel)

            def scale_body(eb, _):
                ew16 = ewv[j, pl.ds(eb * 16, 16)]
                for l in range(16):
                    sc = ew16[l]
                    e = eb * 16 + l
                    for v in range(D // 16):
                        sl = pl.ds(v * 16, 16)
                        buf[e, sl] = buf[e, sl] * sc
                return 0

            lax.fori_loop(0, K // 16, scale_body, 0)

        pltpu.sync_copy(zb, acc.at[pl.ds(s * ZSTRIPE, ZSTRIPE)])
        plsc.subcore_barrier()

        _gather_start(0, gbufa, gsa)
        _gather_start(1, gbufb, gsb)

        def pair_body(t, _):
            a = 2 * t
            b = a + 1
            na = jnp.minimum(a + 2, NCHUNK - 1)
            nb = jnp.minimum(b + 2, NCHUNK - 1)
            _gather_wait(a, gbufa, gsa)
            _process(a, gbufa, colwa)
            _scatter_start(gbufa, colwa, ssa)
            _gather_wait(b, gbufb, gsb)
            _process(b, gbufb, colwb)
            _scatter_start(gbufb, colwb, ssb)
            _scatter_wait(gbufa, colwa, ssa)
            _gather_start(na, gbufa, gsa)
            _scatter_wait(gbufb, colwb, ssb)
            _gather_start(nb, gbufb, gsb)
            return 0

        lax.fori_loop(0, (NCHUNK - 1) // 2, pair_body, 0)

        last = NCHUNK - 1
        _gather_wait(last, gbufa, gsa)
        _process(last, gbufa, colwa)
        _scatter_start(gbufa, colwa, ssa)
        _gather_wait(last, gbufb, gsb)
        _scatter_wait(gbufa, colwa, ssa)

        plsc.subcore_barrier()
        pltpu.sync_copy(
            acc.at[pl.ds(s * OSTRIPE, OSTRIPE)],
            out.at[c, pl.ds(p * PR + s * OSTRIPE, OSTRIPE), :])
        plsc.subcore_barrier()


_NB = 1024
_GRID = (NPAD // _NB,)

_rows = pl.BlockSpec((_NB, D), lambda i: (i, 0))
_spart = pl.BlockSpec((NC, _NB, D), lambda i: (0, i, 0))
_full = pl.BlockSpec((D, D), lambda i: (0, 0))
_bias = pl.BlockSpec((1, D), lambda i: (0, 0))
_degs = pl.BlockSpec((NC, _NB), lambda i: (0, i))


def _dinv_of(degp):
    return lax.rsqrt(degp[0, :] + degp[1, :] + 1.0)[:, None]


def _mm_t(a, w):
    return lax.dot_general(a, w, (((1,), (1,)), ((), ())),
                           preferred_element_type=jnp.float32)


def _tc_pre(x_ref, w1_ref, b1_ref, wg0_ref, degp_ref, g0_ref):
    y1 = _mm_t(x_ref[...], w1_ref[...]) + b1_ref[...]
    g0_ref[...] = _dinv_of(degp_ref[...]) * _mm_t(y1, wg0_ref[...])


def _tc_mid(s3_ref, g0_ref, degp_ref, bg0_ref, wg1_ref, g1_ref):
    dinv = _dinv_of(degp_ref[...])
    y2 = dinv * (s3_ref[0] + s3_ref[1] + g0_ref[...]) + bg0_ref[...]
    g1_ref[...] = dinv * _mm_t(y2, wg1_ref[...])


def _tc_post(s3_ref, g1_ref, degp_ref, bg1_ref, w2_ref, b2_ref, o_ref):
    dinv = _dinv_of(degp_ref[...])
    y3 = dinv * (s3_ref[0] + s3_ref[1] + g1_ref[...]) + bg1_ref[...]
    o_ref[...] = jax.nn.sigmoid(_mm_t(y3, w2_ref[...]) + b2_ref[...])


_gshape = jax.ShapeDtypeStruct((NPAD, D), jnp.float32)

_tc_pre_call = pl.pallas_call(
    _tc_pre,
    grid=_GRID,
    in_specs=[_rows, _full, _bias, _full, _degs],
    out_specs=_rows,
    out_shape=_gshape,
)

_tc_mid_call = pl.pallas_call(
    _tc_mid,
    grid=_GRID,
    in_specs=[_spart, _rows, _degs, _bias, _full],
    out_specs=_rows,
    out_shape=_gshape,
)

_tc_post_call = pl.pallas_call(
    _tc_post,
    grid=_GRID,
    in_specs=[_spart, _rows, _degs, _bias, _full, _bias],
    out_specs=_rows,
    out_shape=jax.ShapeDtypeStruct((N, D), jnp.float32),
)


def kernel(x, edge_index, edge_attr, W_fc1, b_fc1, W_g0, b_g0, W_g1, b_g1,
           W_fc2, b_fc2):
    row3 = edge_index[0].astype(jnp.int32).reshape(NW, NCHUNK, K)
    col3 = edge_index[1].astype(jnp.int32).reshape(NW, NCHUNK, K)
    ew3 = edge_attr.astype(jnp.float32).reshape(NW, NCHUNK, K)

    degp = _sc_degree(col3, ew3)
    g0 = _tc_pre_call(x, W_fc1, b_fc1.reshape(1, D), W_g0, degp)
    s0 = _sc_edge_agg(g0, row3, col3, ew3)
    g1 = _tc_mid_call(s0, g0, degp, b_g0.reshape(1, D), W_g1)
    s1 = _sc_edge_agg(g1, row3, col3, ew3)
    return _tc_post_call(s1, g1, degp, b_g1.reshape(1, D), W_fc2,
                         b_fc2.reshape(1, D))
